# baseline (speedup 1.0000x reference)
import functools

import jax
import jax.numpy as jnp
from jax import lax
from jax.experimental import pallas as pl
from jax.experimental.pallas import tpu as pltpu
from jax.experimental.pallas import tpu_sc as plsc

_NC, _NS = 2, 16
_NW = _NC * _NS
_IW = 128


def _mesh():
    return plsc.VectorSubcoreMesh(
        core_axis_name="c", subcore_axis_name="s",
        num_cores=_NC, num_subcores=_NS)


def _sc_relayout(tt, V, d):
    U = V * d // _IW
    nt = V // _IW
    tail_v = V - nt * _IW
    per_w = nt // _NW
    rem = nt - per_w * _NW

    @functools.partial(
        pl.kernel,
        out_type=jax.ShapeDtypeStruct((U, _IW), jnp.float32),
        mesh=_mesh(),
        scratch_types=[
            pltpu.VMEM((d, _IW), jnp.float32),
            pltpu.VMEM((_IW * d // _IW, _IW), jnp.float32),
        ],
        compiler_params=pltpu.CompilerParams(
            use_tc_tiling_on_sc=True, needs_layout_passes=False),
    )
    def k(tt_hbm, out_hbm, ib, ob):
        wid = lax.axis_index("s") * _NC + lax.axis_index("c")
        extra = jnp.where(wid < rem, 1, 0)
        ct0 = wid * per_w + jnp.minimum(wid, rem)
        nct = per_w + extra

        f16a = lax.iota(jnp.int32, 16)
        f16b = f16a + 16

        def shuffle(src, dst, n_vl):
            for vl in range(n_vl):
                row, col = (32 * vl) // _IW, (32 * vl) % _IW
                va = plsc.load_gather(src, [f16a, jnp.full((16,), vl, jnp.int32)])
                vb = plsc.load_gather(src, [f16b, jnp.full((16,), vl, jnp.int32)])
                dst[row, pl.ds(col, 16)] = va
                dst[row, pl.ds(col + 16, 16)] = vb

        def body(i, carry):
            ct = ct0 + i
            pltpu.sync_copy(tt_hbm.at[:, pl.ds(ct * _IW, _IW)], ib)
            shuffle(ib, ob, _IW)
            pltpu.sync_copy(ob, out_hbm.at[pl.ds(ct * 32, 32)])
            return carry

        lax.fori_loop(0, nct, body, 0)

    return k(tt)


def _sc_gather(idx2d, table, n_rows, d):
    rpw = n_rows // _NW
    nbuf = 4
    nouter = rpw // nbuf

    @functools.partial(
        pl.kernel,
        out_type=jax.ShapeDtypeStruct((n_rows, _IW, d), jnp.float32),
        mesh=_mesh(),
        scratch_types=[
            pltpu.VMEM((rpw, _IW), jnp.int32),
            pltpu.VMEM((nbuf, _IW, d), jnp.float32),
            [pltpu.SemaphoreType.DMA] * nbuf,
            [pltpu.SemaphoreType.DMA] * nbuf,
        ],
        compiler_params=pltpu.CompilerParams(use_tc_tiling_on_sc=False),
    )
    def k(idx_hbm, table_hbm, out_hbm, idx_v, rows_v, gsem, wsem):
        wid = lax.axis_index("s") * _NC + lax.axis_index("c")
        r0 = wid * rpw
        pltpu.sync_copy(idx_hbm.at[pl.ds(r0, rpw)], idx_v)

        for c in range(2):
            pltpu.async_copy(table_hbm.at[idx_v.at[c]], rows_v.at[c], gsem[c])

        def body(i, carry):
            for b in range(nbuf):
                c = i * nbuf + b
                bg = (b + 2) % nbuf

                @pl.when(c >= 2)
                def _():
                    pltpu.make_async_copy(
                        rows_v.at[bg], out_hbm.at[r0 + c - 2], wsem[bg]
                    ).wait()

                @pl.when(c + 2 < rpw)
                def _():
                    pltpu.async_copy(
                        table_hbm.at[idx_v.at[c + 2]], rows_v.at[bg], gsem[bg])

                pltpu.make_async_copy(
                    table_hbm.at[idx_v.at[c]], rows_v.at[b], gsem[b]
                ).wait()
                pltpu.async_copy(rows_v.at[b], out_hbm.at[r0 + c], wsem[b])
            return carry

        lax.fori_loop(0, nouter, body, 0)

        for b in (2, 3):
            c = (nouter - 1) * nbuf + b
            pltpu.make_async_copy(
                rows_v.at[b], out_hbm.at[r0 + c], wsem[b]
            ).wait()

    return k(idx2d, table)


def kernel(cat_idx, table):
    s0, s1 = cat_idx.shape
    V, d = table.shape
    n = s0 * s1
    n_rows = n // _IW
    idx2d = cat_idx.reshape(n_rows, _IW).astype(jnp.int32)
    tlin = _sc_relayout(table.T, V, d)
    vt = (V // _IW) * _IW
    tail = table[vt:].reshape((V - vt) * d // _IW, _IW)
    tlin = lax.dynamic_update_slice(tlin, tail, (vt * d // _IW, 0))
    out = _sc_gather(idx2d, tlin.reshape(V, d), n_rows, d)
    return out.reshape(s0, s1, d)


# speedup vs baseline: 1.1531x; 1.1531x over previous
import functools

import jax
import jax.numpy as jnp
from jax import lax
from jax.experimental import pallas as pl
from jax.experimental.pallas import tpu as pltpu
from jax.experimental.pallas import tpu_sc as plsc

_NC, _NS = 2, 16
_NW = _NC * _NS
_IW = 128


def _mesh():
    return plsc.VectorSubcoreMesh(
        core_axis_name="c", subcore_axis_name="s",
        num_cores=_NC, num_subcores=_NS)


def _sc_relayout(tt, V, d):
    U = V * d // _IW
    nt = V // _IW
    per_w = nt // _NW
    rem = nt - per_w * _NW
    nstep = ((per_w + 2) // 2) * 2

    @functools.partial(
        pl.kernel,
        out_type=jax.ShapeDtypeStruct((U, _IW), jnp.float32),
        mesh=_mesh(),
        scratch_types=[
            pltpu.VMEM((2, d, _IW), jnp.float32),
            pltpu.VMEM((2, d, _IW), jnp.float32),
            [pltpu.SemaphoreType.DMA] * 2,
            [pltpu.SemaphoreType.DMA] * 2,
        ],
        compiler_params=pltpu.CompilerParams(
            use_tc_tiling_on_sc=True, needs_layout_passes=False),
    )
    def k(tt_hbm, out_hbm, ib, ob, isem, osem):
        wid = lax.axis_index("s") * _NC + lax.axis_index("c")
        extra = jnp.where(wid < rem, 1, 0)
        ct0 = wid * per_w + jnp.minimum(wid, rem)
        nct = per_w + extra

        f16a = lax.iota(jnp.int32, 16)
        f16b = f16a + 16

        def in_slice(i):
            return tt_hbm.at[:, pl.ds((ct0 + i) * _IW, _IW)]

        def out_slice(i):
            return out_hbm.at[pl.ds((ct0 + i) * 32, 32)]

        def shuffle(src, dst):
            for vl in range(_IW):
                row, col = (32 * vl) // _IW, (32 * vl) % _IW
                vlv = jnp.full((16,), vl, jnp.int32)
                dst[row, pl.ds(col, 16)] = plsc.load_gather(src, [f16a, vlv])
                dst[row, pl.ds(col + 16, 16)] = plsc.load_gather(src, [f16b, vlv])

        for b in range(2):
            @pl.when(b < nct)
            def _():
                pltpu.async_copy(in_slice(b), ib.at[b], isem[b])

        def body(i2, carry):
            for b in range(2):
                i = i2 * 2 + b

                @pl.when(i < nct)
                def _():
                    pltpu.make_async_copy(in_slice(i), ib.at[b], isem[b]).wait()

                    @pl.when(i >= 2)
                    def _():
                        pltpu.make_async_copy(
                            ob.at[b], out_slice(i - 2), osem[b]).wait()

                    shuffle(ib.at[b], ob.at[b])
                    pltpu.async_copy(ob.at[b], out_slice(i), osem[b])

                    @pl.when(i + 2 < nct)
                    def _():
                        pltpu.async_copy(in_slice(i + 2), ib.at[b], isem[b])
            return carry

        lax.fori_loop(0, nstep // 2, body, 0)

        for b in range(2):
            lb = jnp.where((nct - 1) % 2 == b, nct - 1, nct - 2)

            @pl.when(lb >= 0)
            def _():
                pltpu.make_async_copy(
                    ob.at[b], out_slice(lb), osem[b]).wait()


    return k(tt)


def _sc_gather(idx2d, table, n_rows, d):
    rpw = n_rows // _NW
    nbuf = 4
    nouter = rpw // nbuf

    @functools.partial(
        pl.kernel,
        out_type=jax.ShapeDtypeStruct((n_rows, _IW, d), jnp.float32),
        mesh=_mesh(),
        scratch_types=[
            pltpu.VMEM((rpw, _IW), jnp.int32),
            pltpu.VMEM((nbuf, _IW, d), jnp.float32),
            [pltpu.SemaphoreType.DMA] * nbuf,
            [pltpu.SemaphoreType.DMA] * nbuf,
        ],
        compiler_params=pltpu.CompilerParams(use_tc_tiling_on_sc=False),
    )
    def k(idx_hbm, table_hbm, out_hbm, idx_v, rows_v, gsem, wsem):
        wid = lax.axis_index("s") * _NC + lax.axis_index("c")
        r0 = wid * rpw
        pltpu.sync_copy(idx_hbm.at[pl.ds(r0, rpw)], idx_v)

        for c in range(2):
            pltpu.async_copy(table_hbm.at[idx_v.at[c]], rows_v.at[c], gsem[c])

        def body(i, carry):
            for b in range(nbuf):
                c = i * nbuf + b
                bg = (b + 2) % nbuf

                @pl.when(c >= 2)
                def _():
                    pltpu.make_async_copy(
                        rows_v.at[bg], out_hbm.at[r0 + c - 2], wsem[bg]
                    ).wait()

                @pl.when(c + 2 < rpw)
                def _():
                    pltpu.async_copy(
                        table_hbm.at[idx_v.at[c + 2]], rows_v.at[bg], gsem[bg])

                pltpu.make_async_copy(
                    table_hbm.at[idx_v.at[c]], rows_v.at[b], gsem[b]
                ).wait()
                pltpu.async_copy(rows_v.at[b], out_hbm.at[r0 + c], wsem[b])
            return carry

        lax.fori_loop(0, nouter, body, 0)

        for b in (2, 3):
            c = (nouter - 1) * nbuf + b
            pltpu.make_async_copy(
                rows_v.at[b], out_hbm.at[r0 + c], wsem[b]
            ).wait()

    return k(idx2d, table)


def kernel(cat_idx, table):
    s0, s1 = cat_idx.shape
    V, d = table.shape
    n = s0 * s1
    n_rows = n // _IW
    idx2d = cat_idx.reshape(n_rows, _IW).astype(jnp.int32)
    tlin = _sc_relayout(table.T, V, d)
    vt = (V // _IW) * _IW
    tail = table[vt:].reshape((V - vt) * d // _IW, _IW)
    tlin = lax.dynamic_update_slice(tlin, tail, (vt * d // _IW, 0))
    out = _sc_gather(idx2d, tlin.reshape(V, d), n_rows, d)
    return out.reshape(s0, s1, d)


# speedup vs baseline: 1.7425x; 1.5112x over previous
import functools

import jax
import jax.numpy as jnp
from jax import lax
from jax.experimental import pallas as pl
from jax.experimental.pallas import tpu as pltpu
from jax.experimental.pallas import tpu_sc as plsc

_NC, _NS = 2, 16
_NW = _NC * _NS
_IW = 128


def _mesh():
    return plsc.VectorSubcoreMesh(
        core_axis_name="c", subcore_axis_name="s",
        num_cores=_NC, num_subcores=_NS)


def _sc_relayout(tt, V, d):
    U = V * d // _IW
    nt = V // _IW
    per_w = nt // _NW
    rem = nt - per_w * _NW
    nstep = ((per_w + 2) // 2) * 2

    @functools.partial(
        pl.kernel,
        out_type=jax.ShapeDtypeStruct((U, _IW), jnp.float32),
        mesh=_mesh(),
        scratch_types=[
            pltpu.VMEM((2, d, _IW), jnp.float32),
            pltpu.VMEM((2, d, _IW), jnp.float32),
            [pltpu.SemaphoreType.DMA] * 2,
            [pltpu.SemaphoreType.DMA] * 2,
        ],
        compiler_params=pltpu.CompilerParams(
            use_tc_tiling_on_sc=True, needs_layout_passes=False),
    )
    def k(tt_hbm, out_hbm, ib, ob, isem, osem):
        wid = lax.axis_index("s") * _NC + lax.axis_index("c")
        extra = jnp.where(wid < rem, 1, 0)
        ct0 = wid * per_w + jnp.minimum(wid, rem)
        nct = per_w + extra

        f16a = lax.iota(jnp.int32, 16)
        z16 = f16a * 0
        fa128 = f16a * _IW
        fb128 = fa128 + 16 * _IW

        def in_slice(i):
            return tt_hbm.at[:, pl.ds((ct0 + i) * _IW, _IW)]

        def out_slice(i):
            return out_hbm.at[pl.ds((ct0 + i) * 32, 32)]

        def shuffle(src, dst):
            @plsc.parallel_loop(0, _IW, step=1, unroll=8)
            def _(vl):
                row = vl // 4
                col = (vl % 4) * 32
                dst[row, pl.ds(col, 16)] = plsc.load_gather(
                    src, [z16, fa128 + vl])
                dst[row, pl.ds(col + 16, 16)] = plsc.load_gather(
                    src, [z16, fb128 + vl])

        for b in range(2):
            @pl.when(b < nct)
            def _():
                pltpu.async_copy(in_slice(b), ib.at[b], isem[b])

        def body(i2, carry):
            for b in range(2):
                i = i2 * 2 + b

                @pl.when(i < nct)
                def _():
                    pltpu.make_async_copy(in_slice(i), ib.at[b], isem[b]).wait()

                    @pl.when(i >= 2)
                    def _():
                        pltpu.make_async_copy(
                            ob.at[b], out_slice(i - 2), osem[b]).wait()

                    shuffle(ib.at[b], ob.at[b])
                    pltpu.async_copy(ob.at[b], out_slice(i), osem[b])

                    @pl.when(i + 2 < nct)
                    def _():
                        pltpu.async_copy(in_slice(i + 2), ib.at[b], isem[b])
            return carry

        lax.fori_loop(0, nstep // 2, body, 0)

        for b in range(2):
            lb = jnp.where((nct - 1) % 2 == b, nct - 1, nct - 2)

            @pl.when(lb >= 0)
            def _():
                pltpu.make_async_copy(
                    ob.at[b], out_slice(lb), osem[b]).wait()


    return k(tt)


def _sc_gather(idx2d, table, n_rows, d):
    rpw = n_rows // _NW
    nbuf = 4
    nouter = rpw // nbuf

    @functools.partial(
        pl.kernel,
        out_type=jax.ShapeDtypeStruct((n_rows, _IW, d), jnp.float32),
        mesh=_mesh(),
        scratch_types=[
            pltpu.VMEM((rpw, _IW), jnp.int32),
            pltpu.VMEM((nbuf, _IW, d), jnp.float32),
            [pltpu.SemaphoreType.DMA] * nbuf,
            [pltpu.SemaphoreType.DMA] * nbuf,
        ],
        compiler_params=pltpu.CompilerParams(use_tc_tiling_on_sc=False),
    )
    def k(idx_hbm, table_hbm, out_hbm, idx_v, rows_v, gsem, wsem):
        wid = lax.axis_index("s") * _NC + lax.axis_index("c")
        r0 = wid * rpw
        pltpu.sync_copy(idx_hbm.at[pl.ds(r0, rpw)], idx_v)

        for c in range(2):
            pltpu.async_copy(table_hbm.at[idx_v.at[c]], rows_v.at[c], gsem[c])

        def body(i, carry):
            for b in range(nbuf):
                c = i * nbuf + b
                bg = (b + 2) % nbuf

                @pl.when(c >= 2)
                def _():
                    pltpu.make_async_copy(
                        rows_v.at[bg], out_hbm.at[r0 + c - 2], wsem[bg]
                    ).wait()

                @pl.when(c + 2 < rpw)
                def _():
                    pltpu.async_copy(
                        table_hbm.at[idx_v.at[c + 2]], rows_v.at[bg], gsem[bg])

                pltpu.make_async_copy(
                    table_hbm.at[idx_v.at[c]], rows_v.at[b], gsem[b]
                ).wait()
                pltpu.async_copy(rows_v.at[b], out_hbm.at[r0 + c], wsem[b])
            return carry

        lax.fori_loop(0, nouter, body, 0)

        for b in (2, 3):
            c = (nouter - 1) * nbuf + b
            pltpu.make_async_copy(
                rows_v.at[b], out_hbm.at[r0 + c], wsem[b]
            ).wait()

    return k(idx2d, table)


def kernel(cat_idx, table):
    s0, s1 = cat_idx.shape
    V, d = table.shape
    n = s0 * s1
    n_rows = n // _IW
    idx2d = cat_idx.reshape(n_rows, _IW).astype(jnp.int32)
    tlin = _sc_relayout(table.T, V, d)
    vt = (V // _IW) * _IW
    tail = table[vt:].reshape((V - vt) * d // _IW, _IW)
    tlin = lax.dynamic_update_slice(tlin, tail, (vt * d // _IW, 0))
    out = _sc_gather(idx2d, tlin.reshape(V, d), n_rows, d)
    return out.reshape(s0, s1, d)
